# baseline (speedup 1.0000x reference)
import functools

import jax
import jax.numpy as jnp
from jax import lax
from jax.experimental import pallas as pl
from jax.experimental.pallas import tpu as pltpu
from jax.experimental.pallas import tpu_sc as plsc

MAXLEN = 200
EMBED = 32
BATCH = 4096
NC, NS = 2, 16
NW = NC * NS
BT = 128
EG = EMBED // 8
SG = 4
NG = MAXLEN // SG
NPAIR = NG // 2

_mesh = plsc.VectorSubcoreMesh(core_axis_name="c", subcore_axis_name="s")


@functools.partial(
    pl.kernel,
    mesh=_mesh,
    out_type=jax.ShapeDtypeStruct((MAXLEN, EG, NW, 8, BT), jnp.float32),
    compiler_params=pltpu.CompilerParams(
        use_tc_tiling_on_sc=False, needs_layout_passes=False),
    scratch_types=[
        pltpu.VMEM((SG, BT), jnp.int32),
        pltpu.VMEM((SG, BT), jnp.int32),
        pltpu.VMEM((SG, BT, EMBED), jnp.float32),
        pltpu.VMEM((SG, BT, EMBED), jnp.float32),
        pltpu.VMEM((SG, EG, 8, BT), jnp.float32),
        pltpu.VMEM((SG, EG, 8, BT), jnp.float32),
        pltpu.VMEM((MAXLEN, EMBED), jnp.float32),
        pltpu.SemaphoreType.DMA,
        pltpu.SemaphoreType.DMA,
        pltpu.SemaphoreType.DMA,
        pltpu.SemaphoreType.DMA,
    ],
)
def _embed_kernel(xt_hbm, tok_hbm, pos_hbm, out_hbm,
                  idx_a, idx_b, rows_a, rows_b, outt_a, outt_b, pos_v,
                  sem_idx, sem_g, sem_oa, sem_ob):
    w = lax.axis_index("s") * NC + lax.axis_index("c")
    c0 = w * BT
    iota16 = lax.iota(jnp.int32, 16)

    def idx_src(sg):
        return xt_hbm.at[pl.ds(sg * SG, SG), pl.ds(c0, BT)]

    def fire_gathers(idx_x, rows_x):
        for si in range(SG):
            pltpu.async_copy(tok_hbm.at[idx_x.at[si]], rows_x.at[si], sem_g)

    def wait_gathers(idx_x, rows_x):
        for si in range(SG):
            pltpu.make_async_copy(
                tok_hbm.at[idx_x.at[si]], rows_x.at[si], sem_g).wait()

    def fire_outs(outt_x, sg, sem_o):
        for si in range(SG):
            for g in range(EG):
                pltpu.async_copy(
                    outt_x.at[si, g], out_hbm.at[sg * SG + si, g, w], sem_o)

    def drain_outs(outt_x, sg, sem_o):
        for si in range(SG):
            for g in range(EG):
                pltpu.make_async_copy(
                    outt_x.at[si, g], out_hbm.at[sg * SG + si, g, w],
                    sem_o).wait()

    def transpose_group(rows_x, outt_x, sg):
        for si in range(SG):
            s = sg * SG + si
            si_v = jnp.full((16,), si, jnp.int32)
            s_v = jnp.full((16,), s, jnp.int32)

            def e_body(e, c2):
                g = lax.shift_right_logical(e, 2 + 1)
                r = lax.bitwise_and(e, 7)
                e_v = jnp.full((16,), e, jnp.int32)
                pv = plsc.load_gather(pos_v, [s_v, e_v])
                for t16 in range(BT // 16):
                    t_v = iota16 + (t16 * 16)
                    v = plsc.load_gather(rows_x, [si_v, t_v, e_v])
                    outt_x[si, g, r, pl.ds(t16 * 16, 16)] = v + pv
                return c2

            lax.fori_loop(0, EMBED, e_body, 0)

    pltpu.sync_copy(pos_hbm, pos_v)
    pltpu.sync_copy(idx_src(0), idx_a)
    fire_gathers(idx_a, rows_a)
    pltpu.async_copy(idx_src(1), idx_b, sem_idx)

    def pair_body(p, carry):
        for b in range(2):
            sg = p * 2 + b
            idx_x, rows_x, outt_x, sem_o = (
                (idx_a, rows_a, outt_a, sem_oa) if b == 0
                else (idx_b, rows_b, outt_b, sem_ob))
            idx_y, rows_y = (idx_b, rows_b) if b == 0 else (idx_a, rows_a)

            wait_gathers(idx_x, rows_x)

            @pl.when(sg + 1 < NG)
            def _():
                pltpu.make_async_copy(idx_src(sg + 1), idx_y, sem_idx).wait()
                fire_gathers(idx_y, rows_y)

            @pl.when(sg + 2 < NG)
            def _():
                pltpu.async_copy(idx_src(sg + 2), idx_x, sem_idx)

            @pl.when(sg >= 2)
            def _():
                drain_outs(outt_x, sg - 2, sem_o)

            transpose_group(rows_x, outt_x, sg)
            fire_outs(outt_x, sg, sem_o)
        return carry

    lax.fori_loop(0, NPAIR, pair_body, 0)
    drain_outs(outt_a, NG - 2, sem_oa)
    drain_outs(outt_b, NG - 1, sem_ob)


def kernel(x, token_table, pos_table):
    xt = jnp.transpose(x.astype(jnp.int32))
    out5 = _embed_kernel(xt, token_table, pos_table)
    out = out5.transpose(0, 1, 3, 2, 4).reshape(MAXLEN, EMBED, BATCH)
    return out.transpose(2, 0, 1)


# speedup vs baseline: 1.7744x; 1.7744x over previous
import functools

import jax
import jax.numpy as jnp
from jax import lax
from jax.experimental import pallas as pl
from jax.experimental.pallas import tpu as pltpu
from jax.experimental.pallas import tpu_sc as plsc

MAXLEN = 200
EMBED = 32
BATCH = 4096
NC, NS = 2, 16
NW = NC * NS
BT = 128
EG = EMBED // 8
SG = 4
NG = MAXLEN // SG
NPAIR = NG // 2
BTP = BT + 1

_mesh = plsc.VectorSubcoreMesh(core_axis_name="c", subcore_axis_name="s")


@functools.partial(
    pl.kernel,
    mesh=_mesh,
    out_type=jax.ShapeDtypeStruct((MAXLEN, EG, NW, 8, BT), jnp.float32),
    compiler_params=pltpu.CompilerParams(
        use_tc_tiling_on_sc=False, needs_layout_passes=False),
    scratch_types=[
        pltpu.VMEM((SG, BT), jnp.int32),
        pltpu.VMEM((SG, BT), jnp.int32),
        pltpu.VMEM((SG, BT, EMBED), jnp.float32),
        pltpu.VMEM((SG, BT, EMBED), jnp.float32),
        pltpu.VMEM((SG * EMBED, BTP), jnp.float32),
        pltpu.VMEM((SG * EMBED, BTP), jnp.float32),
        pltpu.VMEM((MAXLEN, EMBED), jnp.float32),
        pltpu.SemaphoreType.DMA,
        pltpu.SemaphoreType.DMA,
        pltpu.SemaphoreType.DMA,
        pltpu.SemaphoreType.DMA,
    ],
)
def _embed_kernel(xt_hbm, tok_hbm, pos_hbm, out_hbm,
                  idx_a, idx_b, rows_a, rows_b, outt_a, outt_b, pos_v,
                  sem_idx, sem_g, sem_oa, sem_ob):
    w = lax.axis_index("s") * NC + lax.axis_index("c")
    c0 = w * BT
    iota16 = lax.iota(jnp.int32, 16)

    def idx_src(sg):
        return xt_hbm.at[pl.ds(sg * SG, SG), pl.ds(c0, BT)]

    def fire_gathers(idx_x, rows_x):
        for si in range(SG):
            pltpu.async_copy(tok_hbm.at[idx_x.at[si]], rows_x.at[si], sem_g)

    def wait_gathers(idx_x, rows_x):
        for si in range(SG):
            pltpu.make_async_copy(
                tok_hbm.at[idx_x.at[si]], rows_x.at[si], sem_g).wait()

    def fire_outs(outt_x, sg, sem_o):
        for si in range(SG):
            for g in range(EG):
                pltpu.async_copy(
                    outt_x.at[pl.ds(si * EMBED + g * 8, 8), pl.ds(0, BT)],
                    out_hbm.at[sg * SG + si, g, w], sem_o)

    def drain_outs(outt_x, sg, sem_o):
        for si in range(SG):
            for g in range(EG):
                pltpu.make_async_copy(
                    outt_x.at[pl.ds(si * EMBED + g * 8, 8), pl.ds(0, BT)],
                    out_hbm.at[sg * SG + si, g, w], sem_o).wait()

    def transpose_group(rows_x, outt_x, sg):
        for si in range(SG):
            s = sg * SG + si
            p0 = pos_v[s, pl.ds(0, 16)]
            p1 = pos_v[s, pl.ds(16, 16)]
            r0 = iota16 + (si * EMBED)
            r1 = r0 + 16

            def t_body(t4, c2):
                for u in range(4):
                    t = t4 * 4 + u
                    t_v = jnp.full((16,), t, jnp.int32)
                    v0 = rows_x[si, t, pl.ds(0, 16)] + p0
                    v1 = rows_x[si, t, pl.ds(16, 16)] + p1
                    plsc.store_scatter(outt_x, [r0, t_v], v0)
                    plsc.store_scatter(outt_x, [r1, t_v], v1)
                return c2

            lax.fori_loop(0, BT // 4, t_body, 0)

    pltpu.sync_copy(pos_hbm, pos_v)
    pltpu.sync_copy(idx_src(0), idx_a)
    fire_gathers(idx_a, rows_a)
    pltpu.async_copy(idx_src(1), idx_b, sem_idx)

    def pair_body(p, carry):
        for b in range(2):
            sg = p * 2 + b
            idx_x, rows_x, outt_x, sem_o = (
                (idx_a, rows_a, outt_a, sem_oa) if b == 0
                else (idx_b, rows_b, outt_b, sem_ob))
            idx_y, rows_y = (idx_b, rows_b) if b == 0 else (idx_a, rows_a)

            wait_gathers(idx_x, rows_x)

            @pl.when(sg + 1 < NG)
            def _():
                pltpu.make_async_copy(idx_src(sg + 1), idx_y, sem_idx).wait()
                fire_gathers(idx_y, rows_y)

            @pl.when(sg + 2 < NG)
            def _():
                pltpu.async_copy(idx_src(sg + 2), idx_x, sem_idx)

            @pl.when(sg >= 2)
            def _():
                drain_outs(outt_x, sg - 2, sem_o)

            transpose_group(rows_x, outt_x, sg)
            fire_outs(outt_x, sg, sem_o)
        return carry

    lax.fori_loop(0, NPAIR, pair_body, 0)
    drain_outs(outt_a, NG - 2, sem_oa)
    drain_outs(outt_b, NG - 1, sem_ob)


def kernel(x, token_table, pos_table):
    xt = jnp.transpose(x.astype(jnp.int32))
    out5 = _embed_kernel(xt, token_table, pos_table)
    out = out5.transpose(0, 1, 3, 2, 4).reshape(MAXLEN, EMBED, BATCH)
    return out.transpose(2, 0, 1)
